# Initial kernel scaffold; baseline (speedup 1.0000x reference)
import jax
import jax.numpy as jnp
from jax.experimental import pallas as pl


def kernel(inputs, real_table, imag_table):
    raise NotImplementedError("write your pallas kernel here")


# speedup vs baseline: 1.6486x; 1.6486x over previous
import functools

import jax
import jax.numpy as jnp
from jax import lax
from jax.experimental import pallas as pl
from jax.experimental.pallas import tpu as pltpu
from jax.experimental.pallas import tpu_sc as plsc

B, H = 4096, 200
D = 128
N = B * H
NC, NS, L = 2, 16, 16
NW = NC * NS
RPW = N // NW
CH = 128
NCHUNK = RPW // CH

HALF_PI = 1.5707963267948966
PI = 3.141592653589793
A0 = 0.9999966199359364
A1 = -0.33305310408451655
A2 = 0.19617204791303705
A3 = -0.12292207674757487
A4 = 0.05960017845012795
A5 = -0.014406472473067849


@functools.partial(
    pl.kernel,
    out_type=jax.ShapeDtypeStruct((N, 2, D), jnp.float32),
    mesh=plsc.VectorSubcoreMesh(core_axis_name="c", subcore_axis_name="s"),
    scratch_types=[
        pltpu.VMEM((CH,), jnp.int32),
        pltpu.VMEM((CH, D), jnp.float32),
        pltpu.VMEM((CH, D), jnp.float32),
        pltpu.SemaphoreType.DMA,
        pltpu.SemaphoreType.DMA,
    ],
)
def _qemb(idx_hbm, real_hbm, imag_hbm, out_hbm, idx_v, re_v, im_v, sem_r, sem_i):
    wid = lax.axis_index("s") * NC + lax.axis_index("c")
    wbase = wid * RPW

    def chunk_body(ci, carry):
        base = wbase + ci * CH
        pltpu.sync_copy(idx_hbm.at[pl.ds(base, CH)], idx_v)
        cp_r = pltpu.async_copy(real_hbm.at[idx_v], re_v, sem_r)
        cp_i = pltpu.async_copy(imag_hbm.at[idx_v], im_v, sem_i)
        cp_r.wait()
        cp_i.wait()

        def row_body(row, c2):
            for l in range(D // L):
                sl = pl.ds(l * L, L)
                r = re_v[row, sl]
                i = im_v[row, sl]
                x = r * r + i * i
                xi = lax.bitcast_convert_type(x, jnp.int32)
                y = lax.bitcast_convert_type(
                    jnp.int32(0x5F3759DF) - (xi >> 1), jnp.float32)
                y = y * (1.5 - 0.5 * x * y * y)
                y = y * (1.5 - 0.5 * x * y * y)
                y = y * (1.5 - 0.5 * x * y * y)
                mag = jnp.where(x > 0.0, x * y, 0.0)

                ax = jnp.abs(r)
                ay = jnp.abs(i)
                mx = jnp.maximum(ax, ay)
                mn = jnp.minimum(ax, ay)
                den = jnp.where(mx > 0.0, mx, 1.0)
                t = mn / den
                u = t * t
                p = A5
                p = p * u + A4
                p = p * u + A3
                p = p * u + A2
                p = p * u + A1
                p = p * u + A0
                ph = p * t
                ph = jnp.where(ay > ax, HALF_PI - ph, ph)
                ph = jnp.where(r < 0.0, PI - ph, ph)
                ph = jnp.where(i < 0.0, -ph, ph)

                re_v[row, sl] = mag
                im_v[row, sl] = ph
            return c2

        lax.fori_loop(0, CH, row_body, 0, unroll=False)
        pltpu.sync_copy(re_v, out_hbm.at[pl.ds(base, CH), 0])
        pltpu.sync_copy(im_v, out_hbm.at[pl.ds(base, CH), 1])
        return carry

    lax.fori_loop(0, NCHUNK, chunk_body, 0, unroll=False)


def kernel(inputs, real_table, imag_table):
    idx = inputs.reshape(N).astype(jnp.int32)
    out = _qemb(idx, real_table, imag_table)
    return out.reshape(B, H, 2 * D)
